# Initial kernel scaffold; baseline (speedup 1.0000x reference)
import jax
import jax.numpy as jnp
from jax.experimental import pallas as pl


def kernel(q, c, emb):
    raise NotImplementedError("write your pallas kernel here")


# speedup vs baseline: 2.1430x; 2.1430x over previous
import functools

import jax
import jax.numpy as jnp
from jax import lax
from jax.experimental import pallas as pl
from jax.experimental.pallas import tpu as pltpu
from jax.experimental.pallas import tpu_sc as plsc

N_CTX1 = 6
N_CTX2 = 20
CHUNK = 15

B = 16
LQ = 20
N = 8192
LC = 60
D = 64
N_CHUNKS = (N_CTX1 * LC) // CHUNK

NC = 2
NS = 16
NW = NC * NS

_MESH = dict(core_axis_name="c", subcore_axis_name="s",
             num_cores=NC, num_subcores=NS)
_SC_PARAMS = pltpu.CompilerParams(use_tc_tiling_on_sc=False)


def _wid():
    return lax.axis_index("s") * NC + lax.axis_index("c")


@functools.partial(
    pl.kernel,
    out_type=jax.ShapeDtypeStruct((B * LQ, D), jnp.float32),
    mesh=plsc.VectorSubcoreMesh(**_MESH),
    compiler_params=_SC_PARAMS,
    scratch_types=[
        pltpu.VMEM((LQ,), jnp.int32),
        pltpu.VMEM((LQ, D), jnp.float32),
        pltpu.SemaphoreType.DMA,
    ],
)
def _gq_kernel(q_hbm, emb_hbm, gq_hbm, tok_v, rows_v, sem):
    w = _wid()

    @pl.when(w < B)
    def _():
        pltpu.sync_copy(q_hbm.at[w], tok_v)
        pltpu.async_copy(emb_hbm.at[tok_v], rows_v, sem).wait()
        pltpu.sync_copy(rows_v, gq_hbm.at[pl.ds(w * LQ, LQ)])


_GROWS = 512
_TPW = (N * LC) // NW
_GSTEP = _TPW // _GROWS


@functools.partial(
    pl.kernel,
    out_type=jax.ShapeDtypeStruct((N * LC, D), jnp.float32),
    mesh=plsc.VectorSubcoreMesh(**_MESH),
    compiler_params=_SC_PARAMS,
    scratch_types=[
        pltpu.VMEM((_GSTEP, _GROWS), jnp.int32),
        pltpu.VMEM((_GROWS, D), jnp.float32),
        pltpu.VMEM((_GROWS, D), jnp.float32),
        pltpu.SemaphoreType.DMA,
        pltpu.SemaphoreType.DMA,
        pltpu.SemaphoreType.DMA,
        pltpu.SemaphoreType.DMA,
    ],
)
def _gc_kernel(c3d_hbm, emb_hbm, gc_hbm, idx_v, buf0, buf1,
               sg0, sg1, sw0, sw1):
    w = _wid()
    base = pl.multiple_of(w * _TPW, 8)
    pltpu.sync_copy(c3d_hbm.at[w], idx_v)

    bufs = (buf0, buf1)
    sgs = (sg0, sg1)
    sws = (sw0, sw1)
    gh = {}
    wh = {}
    for g in range(_GSTEP):
        b = g % 2
        if g >= 2:
            wh[g - 2].wait()
        gh[g] = pltpu.async_copy(
            emb_hbm.at[idx_v.at[g]], bufs[b], sgs[b])
        if g >= 1:
            gh[g - 1].wait()
            wh[g - 1] = pltpu.async_copy(
                bufs[(g - 1) % 2],
                gc_hbm.at[pl.ds(base + (g - 1) * _GROWS, _GROWS)],
                sws[(g - 1) % 2])
    gh[_GSTEP - 1].wait()
    wh[_GSTEP - 1] = pltpu.async_copy(
        bufs[(_GSTEP - 1) % 2],
        gc_hbm.at[pl.ds(base + (_GSTEP - 1) * _GROWS, _GROWS)],
        sws[(_GSTEP - 1) % 2])
    wh[_GSTEP - 2].wait()
    wh[_GSTEP - 1].wait()


_SPW = (B * N_CTX1) // NW


@functools.partial(
    pl.kernel,
    out_type=jax.ShapeDtypeStruct((B * N_CTX1 * LC, D), jnp.float32),
    mesh=plsc.VectorSubcoreMesh(**_MESH),
    compiler_params=_SC_PARAMS,
    scratch_types=[
        pltpu.VMEM((B * N_CTX1,), jnp.int32),
        pltpu.VMEM((B * N_CTX1, D), jnp.int32),
        pltpu.VMEM((D, D), jnp.float32),
        pltpu.SemaphoreType.DMA,
    ],
)
def _gch_kernel(top1_hbm, c64_hbm, emb_hbm, gch_hbm, idx_v, crows_v,
                erows_v, sem):
    w = _wid()
    pltpu.sync_copy(top1_hbm, idx_v)
    pltpu.async_copy(c64_hbm.at[idx_v], crows_v, sem).wait()
    for j in range(_SPW):
        s = w * _SPW + j
        pltpu.async_copy(emb_hbm.at[crows_v.at[s]], erows_v, sem).wait()
        pltpu.sync_copy(erows_v.at[pl.ds(0, LC)],
                        gch_hbm.at[pl.ds(s * LC, LC)])


def _s1_body(qe_ref, ce_ref, idx_ref):
    s = lax.dot_general(qe_ref[...], ce_ref[...], (((1,), (1,)), ((), ())),
                        preferred_element_type=jnp.float32)
    m = jnp.max(s, axis=1, keepdims=True)
    sh = s - m
    e = jnp.exp(sh)
    acc = e[:, 0:512]
    for t in range(1, 16):
        acc = acc + e[:, t * 512:(t + 1) * 512]
    width = 512
    while width > 1:
        h = width // 2
        acc = acc[:, :h] + acc[:, h:width]
        width = h
    ls = sh - jnp.log(acc)
    iota = lax.broadcasted_iota(jnp.int32, (B, N), 1)
    for k in range(N_CTX1):
        mx = jnp.max(ls, axis=1, keepdims=True)
        idxk = jnp.min(jnp.where(ls == mx, iota, N), axis=1)
        idx_ref[:, k] = idxk
        ls = jnp.where(iota == idxk[:, None], -jnp.inf, ls)


def _s1_call(qe, ce):
    return pl.pallas_call(
        _s1_body,
        out_shape=jax.ShapeDtypeStruct((B, N_CTX1), jnp.int32),
    )(qe, ce)


def _s2_body(qe_ref, ch_ref, out_ref):
    qe = qe_ref[...]
    ch = ch_ref[...]
    rows = []
    for b in range(B):
        rows.append(lax.dot_general(
            qe[b:b + 1, :], ch[b * N_CHUNKS:(b + 1) * N_CHUNKS, :],
            (((1,), (1,)), ((), ())), preferred_element_type=jnp.float32))
    out_ref[...] = jnp.concatenate(rows, axis=0)


def _s2_call(qe, chunk_flat):
    return pl.pallas_call(
        _s2_body,
        out_shape=jax.ShapeDtypeStruct((B, N_CHUNKS), jnp.float32),
    )(qe, chunk_flat)


def _top20_body(ls_ref, idx_ref):
    ls = ls_ref[...]
    iota = lax.broadcasted_iota(jnp.int32, (B, N_CHUNKS), 1)
    for k in range(N_CTX2):
        mx = jnp.max(ls, axis=1, keepdims=True)
        idxk = jnp.min(jnp.where(ls == mx, iota, N_CHUNKS), axis=1)
        idx_ref[:, k] = idxk
        ls = jnp.where(iota == idxk[:, None], -jnp.inf, ls)


def _top20_call(ls2):
    return pl.pallas_call(
        _top20_body,
        out_shape=jax.ShapeDtypeStruct((B, N_CTX2), jnp.int32),
    )(ls2)


def _final_body(idx_ref, chunk_ref, out_ref):
    for b in range(B):
        for k in range(N_CTX2):
            r = b * N_CHUNKS + idx_ref[b, k]
            out_ref[b, pl.ds(k, 1), :] = chunk_ref[pl.ds(r, 1), :]


def _final_call(chunk_flat, idx2):
    return pl.pallas_call(
        _final_body,
        in_specs=[
            pl.BlockSpec(memory_space=pltpu.MemorySpace.SMEM),
            pl.BlockSpec(memory_space=pltpu.MemorySpace.VMEM),
        ],
        out_shape=jax.ShapeDtypeStruct((B, N_CTX2, D), jnp.float32),
    )(idx2, chunk_flat)


def kernel(q, c, emb):
    q = q.astype(jnp.int32)
    c = c.astype(jnp.int32)
    gq = _gq_kernel(q, emb)
    qe = gq.reshape(B, LQ, D).mean(axis=1)
    gc = _gc_kernel(c.reshape(NW, _GSTEP, _GROWS), emb)
    ce = gc.reshape(N, LC, D).mean(axis=1)
    top1 = _s1_call(qe, ce)
    c64 = jnp.pad(c, ((0, 0), (0, D - LC)))
    gch = _gch_kernel(top1.reshape(-1), c64, emb)
    chunk_e = gch.reshape(B, N_CHUNKS, CHUNK, D).mean(axis=2)
    chunk_flat = chunk_e.reshape(B * N_CHUNKS, D)
    scores2 = _s2_call(qe, chunk_flat)
    ls2 = jax.nn.log_softmax(scores2, axis=1)
    idx2 = _top20_call(ls2)
    return _final_call(chunk_flat, idx2)
